# baseline (speedup 1.0000x reference)
import jax
import jax.numpy as jnp
from jax import lax
from jax.experimental import pallas as pl
from jax.experimental.pallas import tpu as pltpu
from jax.experimental.pallas import tpu_sc as plsc

EMB = 64
T = 200
B = 4096
NC, NS, L = 2, 16, 16
NW = NC * NS
BB = B // NW
NBUF = 4
NOB = 2
QV = EMB // L


def _body(x_hbm, tab_hbm, pos_hbm, out_hbm,
          xblk, posv,
          buf0, buf1, buf2, buf3, tb0, tb1,
          gs0, gs1, gs2, gs3, os0, os1):
    bufs = (buf0, buf1, buf2, buf3)
    tbufs = (tb0, tb1)
    gsems = (gs0, gs1, gs2, gs3)
    osems = (os0, os1)

    w = lax.axis_index("s") * NC + lax.axis_index("c")

    pltpu.sync_copy(x_hbm.at[w], xblk)
    pltpu.sync_copy(pos_hbm, posv)

    iota = lax.iota(jnp.int32, L)
    colv = iota * BB

    def gather_start(t, b):
        pltpu.async_copy(tab_hbm.at[xblk.at[t]], bufs[b], gsems[b])

    def gather_wait(t, b):
        pltpu.make_async_copy(tab_hbm.at[xblk.at[t]], bufs[b],
                              gsems[b]).wait()

    def out_start(t, tb):
        for cg in range(8):
            pltpu.async_copy(tbufs[tb].at[pl.ds(cg * 8 * BB, 8 * BB)],
                             out_hbm.at[t, cg, w], osems[tb])

    def out_wait(t, tb):
        for cg in range(8):
            pltpu.make_async_copy(tbufs[tb].at[pl.ds(cg * 8 * BB, 8 * BB)],
                                  out_hbm.at[t, cg, w], osems[tb]).wait()

    for b in range(NBUF):
        gather_start(b, b)

    def group(g, _):
        for b in range(NBUF):
            t = g * NBUF + b
            tb = b % NOB
            gather_wait(t, b)

            pq = [posv[t, pl.ds(q * L, L)] for q in range(QV)]

            if b < NOB:
                @pl.when(g > 0)
                def _():
                    out_wait(t, tb)
            else:
                out_wait(t, tb)

            @plsc.parallel_loop(0, BB, 1, unroll=4)
            def _row(r, _b=b, _tb=tb, _pq=pq):
                for q in range(QV):
                    v = bufs[_b][r, pl.ds(q * L, L)] + _pq[q]
                    plsc.store_scatter(tbufs[_tb],
                                       [colv + (q * (L * BB) + r)], v)

            out_start(t, tb)

            @pl.when(g < (T // NBUF) - 1)
            def _():
                gather_start(t + NBUF, b)
        return 0

    lax.fori_loop(0, T // NBUF, group, 0)

    for b in range(NOB):
        out_wait(T - NOB + b, b % NOB)


@jax.jit
def kernel(x, table, pos_emb):
    xw = x.T.reshape(T, NW, BB).transpose(1, 0, 2)
    trow = jnp.concatenate([table[0::2], table[1::2]], axis=1)
    trow = trow.reshape(table.shape[0], EMB)
    post = pos_emb[:T]

    kfn = pl.kernel(
        _body,
        out_type=jax.ShapeDtypeStruct((T, 8, NW, 8 * BB), jnp.float32),
        compiler_params=pltpu.CompilerParams(use_tc_tiling_on_sc=False,
                                             needs_layout_passes=False),
        mesh=plsc.VectorSubcoreMesh(
            core_axis_name="c", subcore_axis_name="s",
            num_cores=NC, num_subcores=NS),
        scratch_types=[
            pltpu.VMEM((T, BB), jnp.int32),
            pltpu.VMEM((T, EMB), jnp.float32),
            pltpu.VMEM((BB, EMB), jnp.float32),
            pltpu.VMEM((BB, EMB), jnp.float32),
            pltpu.VMEM((BB, EMB), jnp.float32),
            pltpu.VMEM((BB, EMB), jnp.float32),
            pltpu.VMEM((EMB * BB,), jnp.float32),
            pltpu.VMEM((EMB * BB,), jnp.float32),
        ] + [pltpu.SemaphoreType.DMA] * (NBUF + NOB),
    )
    out = kfn(xw, trow, post)
    out = out.reshape(T, 8, NW, 8, BB)
    return out.transpose(2, 4, 0, 1, 3).reshape(B, T, EMB)


# speedup vs baseline: 7.2157x; 7.2157x over previous
import jax
import jax.numpy as jnp
from jax import lax
from jax.experimental import pallas as pl
from jax.experimental.pallas import tpu as pltpu
from jax.experimental.pallas import tpu_sc as plsc

EMB = 64
T = 200
B = 4096
NC, NS, L = 2, 16, 16
NW = NC * NS
BB = B // NW
NBUF = 4
NOB = 2
QV = EMB // L


def _body(x_hbm, tab_hbm, pos_hbm, out_hbm,
          xblk, posv,
          buf0, buf1, buf2, buf3, tb0, tb1,
          gs0, gs1, gs2, gs3, os0, os1):
    bufs = (buf0, buf1, buf2, buf3)
    tbufs = (tb0, tb1)
    gsems = (gs0, gs1, gs2, gs3)
    osems = (os0, os1)

    w = lax.axis_index("s") * NC + lax.axis_index("c")

    pltpu.sync_copy(x_hbm.at[w], xblk)
    pltpu.sync_copy(pos_hbm, posv)

    iota = lax.iota(jnp.int32, L)
    colv = iota * BB

    def gather_start(t, b):
        pltpu.async_copy(tab_hbm.at[xblk.at[t]], bufs[b], gsems[b])

    def gather_wait(t, b):
        pltpu.make_async_copy(tab_hbm.at[xblk.at[t]], bufs[b],
                              gsems[b]).wait()

    def out_start(t, tb):
        for cg in range(8):
            pltpu.async_copy(tbufs[tb].at[pl.ds(cg * 8 * BB, 8 * BB)],
                             out_hbm.at[t, cg, w], osems[tb])

    def out_wait(t, tb):
        for cg in range(8):
            pltpu.make_async_copy(tbufs[tb].at[pl.ds(cg * 8 * BB, 8 * BB)],
                                  out_hbm.at[t, cg, w], osems[tb]).wait()

    for b in range(NBUF):
        gather_start(b, b)

    def group(g, _):
        for b in range(NBUF):
            t = g * NBUF + b
            tb = b % NOB
            gather_wait(t, b)

            pq = [posv[t, pl.ds(q * L, L)] for q in range(QV)]

            if b < NOB:
                @pl.when(g > 0)
                def _():
                    out_wait(t, tb)
            else:
                out_wait(t, tb)

            @plsc.parallel_loop(0, BB, 1, unroll=4)
            def _row(r, _b=b, _tb=tb, _pq=pq):
                for q in range(QV):
                    v = bufs[_b][r, pl.ds(q * L, L)] + _pq[q]
                    plsc.store_scatter(tbufs[_tb],
                                       [colv + (q * (L * BB) + r)], v)

            out_start(t, tb)

            @pl.when(g < (T // NBUF) - 1)
            def _():
                gather_start(t + NBUF, b)
        return 0

    lax.fori_loop(0, T // NBUF, group, 0)

    for b in range(NOB):
        out_wait(T - NOB + b, b % NOB)


@jax.jit
def kernel(x, table, pos_emb):
    xw = x.T.reshape(T, NW, BB).transpose(1, 0, 2)
    trow = table
    post = pos_emb[:T]

    kfn = pl.kernel(
        _body,
        out_type=jax.ShapeDtypeStruct((T, 8, NW, 8 * BB), jnp.float32),
        compiler_params=pltpu.CompilerParams(use_tc_tiling_on_sc=False,
                                             needs_layout_passes=False),
        mesh=plsc.VectorSubcoreMesh(
            core_axis_name="c", subcore_axis_name="s",
            num_cores=NC, num_subcores=NS),
        scratch_types=[
            pltpu.VMEM((T, BB), jnp.int32),
            pltpu.VMEM((T, EMB), jnp.float32),
            pltpu.VMEM((BB, EMB), jnp.float32),
            pltpu.VMEM((BB, EMB), jnp.float32),
            pltpu.VMEM((BB, EMB), jnp.float32),
            pltpu.VMEM((BB, EMB), jnp.float32),
            pltpu.VMEM((EMB * BB,), jnp.float32),
            pltpu.VMEM((EMB * BB,), jnp.float32),
        ] + [pltpu.SemaphoreType.DMA] * (NBUF + NOB),
    )
    out = kfn(xw, trow, post)
    out = out.reshape(T, 8, NW, 8, BB)
    return out.transpose(2, 4, 0, 1, 3).reshape(B, T, EMB)
